# Initial kernel scaffold; baseline (speedup 1.0000x reference)
import functools

import jax
import jax.numpy as jnp
from jax import lax
from jax.experimental import pallas as pl
from jax.experimental.pallas import tpu as pltpu
from jax.experimental.pallas import tpu_sc as plsc

_N_NODES = 10000
_NODE_DIM = 128
_HIDDEN = 64
_NUM_GRAPHS = 256
_N_EDGES = 320000

_NC = 2
_NS = 16
_NW = _NC * _NS
_K = 128
_EPW = 10240
_CH = _EPW // _K
_E_PAD = _NW * _EPW
_SINK = _N_NODES
_N_ACC = 10240
_STRIPE = _N_ACC // _NS
_DEG_W = 16

_BR = 1000
_GRID = _N_NODES // _BR

_PREC = lax.Precision.HIGHEST

_mesh = plsc.VectorSubcoreMesh(core_axis_name="c", subcore_axis_name="s")


def _make_sc_degree(interpret=False):
  @functools.partial(
      pl.kernel,
      out_type=jax.ShapeDtypeStruct((_NC, _N_ACC, _DEG_W), jnp.float32),
      mesh=_mesh,
      scratch_types=[
          pltpu.VMEM((_CH, _K), jnp.int32),
          pltpu.VMEM((_K, _DEG_W), jnp.float32),
          pltpu.VMEM_SHARED((_N_ACC, _DEG_W), jnp.float32),
      ],
      interpret=interpret,
  )
  def sc_degree(dst_hbm, ones_hbm, zeros_hbm, out_hbm, didx, ones_v, acc):
    c = lax.axis_index("c")
    s = lax.axis_index("s")
    wid = c * _NS + s
    pltpu.sync_copy(dst_hbm.at[wid], didx)
    pltpu.sync_copy(ones_hbm, ones_v)
    pltpu.sync_copy(zeros_hbm, acc.at[pl.ds(s * _STRIPE, _STRIPE)])
    plsc.subcore_barrier()

    def chunk(j, carry):
      pltpu.sync_copy(ones_v, acc.at[didx.at[j]], add=True)
      return carry

    lax.fori_loop(0, _CH, chunk, 0)
    plsc.subcore_barrier()
    pltpu.sync_copy(acc.at[pl.ds(s * _STRIPE, _STRIPE)],
                    out_hbm.at[c, pl.ds(s * _STRIPE, _STRIPE)])

  return sc_degree


def _make_sc_aggregate(interpret=False):
  @functools.partial(
      pl.kernel,
      out_type=jax.ShapeDtypeStruct((_NC, _N_ACC, _HIDDEN), jnp.float32),
      mesh=_mesh,
      scratch_types=[
          pltpu.VMEM((_CH, _K), jnp.int32),
          pltpu.VMEM((_CH, _K), jnp.int32),
          pltpu.VMEM((_K, _HIDDEN), jnp.float32),
          pltpu.VMEM_SHARED((_N_ACC, _HIDDEN), jnp.float32),
          pltpu.SemaphoreType.DMA,
      ],
      interpret=interpret,
  )
  def sc_aggregate(src_hbm, dst_hbm, g_hbm, zeros_hbm, out_hbm,
                   sidx, didx, rows, acc, sem):
    c = lax.axis_index("c")
    s = lax.axis_index("s")
    wid = c * _NS + s
    pltpu.sync_copy(src_hbm.at[wid], sidx)
    pltpu.sync_copy(dst_hbm.at[wid], didx)
    pltpu.sync_copy(zeros_hbm, acc.at[pl.ds(s * _STRIPE, _STRIPE)])
    plsc.subcore_barrier()

    def chunk(j, carry):
      pltpu.async_copy(g_hbm.at[sidx.at[j]], rows, sem).wait()
      pltpu.sync_copy(rows, acc.at[didx.at[j]], add=True)
      return carry

    lax.fori_loop(0, _CH, chunk, 0)
    plsc.subcore_barrier()
    pltpu.sync_copy(acc.at[pl.ds(s * _STRIPE, _STRIPE)],
                    out_hbm.at[c, pl.ds(s * _STRIPE, _STRIPE)])

  return sc_aggregate


def _tc_encode_body(deg_ref, x_ref, wn_ref, bn_ref, w1_ref, g1_ref, dinv_ref):
  deg = deg_ref[0, :, 0:1] + deg_ref[1, :, 0:1] + 1.0
  dinv = lax.rsqrt(jnp.maximum(deg, 1.0))
  h0 = jnp.maximum(
      jnp.dot(x_ref[...], wn_ref[...], precision=_PREC) + bn_ref[...], 0.0)
  g1_ref[...] = dinv * jnp.dot(h0, w1_ref[...], precision=_PREC)
  dinv_ref[...] = dinv


def _make_tc_encode(interpret=False):
  return pl.pallas_call(
      _tc_encode_body,
      grid=(_GRID,),
      in_specs=[
          pl.BlockSpec((_NC, _BR, _DEG_W), lambda i: (0, i, 0)),
          pl.BlockSpec((_BR, _NODE_DIM), lambda i: (i, 0)),
          pl.BlockSpec((_NODE_DIM, _HIDDEN), lambda i: (0, 0)),
          pl.BlockSpec((1, _HIDDEN), lambda i: (0, 0)),
          pl.BlockSpec((_HIDDEN, _HIDDEN), lambda i: (0, 0)),
      ],
      out_specs=[
          pl.BlockSpec((_BR, _HIDDEN), lambda i: (i, 0)),
          pl.BlockSpec((_BR, 1), lambda i: (i, 0)),
      ],
      out_shape=[
          jax.ShapeDtypeStruct((_N_NODES, _HIDDEN), jnp.float32),
          jax.ShapeDtypeStruct((_N_NODES, 1), jnp.float32),
      ],
      interpret=interpret,
  )


def _tc_layer_body(ap_ref, g_ref, dinv_ref, b_ref, w_ref, out_ref):
  a = ap_ref[0] + ap_ref[1] + g_ref[...]
  h = jnp.maximum(dinv_ref[...] * a + b_ref[...], 0.0)
  out_ref[...] = dinv_ref[...] * jnp.dot(h, w_ref[...], precision=_PREC)


def _make_tc_layer(interpret=False):
  return pl.pallas_call(
      _tc_layer_body,
      grid=(_GRID,),
      in_specs=[
          pl.BlockSpec((_NC, _BR, _HIDDEN), lambda i: (0, i, 0)),
          pl.BlockSpec((_BR, _HIDDEN), lambda i: (i, 0)),
          pl.BlockSpec((_BR, 1), lambda i: (i, 0)),
          pl.BlockSpec((1, _HIDDEN), lambda i: (0, 0)),
          pl.BlockSpec((_HIDDEN, _HIDDEN), lambda i: (0, 0)),
      ],
      out_specs=pl.BlockSpec((_BR, _HIDDEN), lambda i: (i, 0)),
      out_shape=jax.ShapeDtypeStruct((_N_NODES, _HIDDEN), jnp.float32),
      interpret=interpret,
  )


def _tc_head_body(ap_ref, g_ref, dinv_ref, b_ref, batch_ref,
                  wf1_ref, bf1_ref, wf2_ref, bf2_ref, out_ref, sums, counts):
  i = pl.program_id(0)

  @pl.when(i == 0)
  def _():
    sums[...] = jnp.zeros_like(sums)
    counts[...] = jnp.zeros_like(counts)

  a = ap_ref[0] + ap_ref[1] + g_ref[...]
  h3 = jnp.maximum(dinv_ref[...] * a + b_ref[...], 0.0)
  iota = lax.broadcasted_iota(jnp.float32, (1, _NUM_GRAPHS), 1)
  onehot = (batch_ref[...] == iota).astype(jnp.float32)
  dn = (((0,), (0,)), ((), ()))
  sums[...] += lax.dot_general(onehot, h3, dn, precision=_PREC)
  counts[...] += lax.dot_general(
      onehot, jnp.ones((_BR, 1), jnp.float32), dn, precision=_PREC)

  @pl.when(i == _GRID - 1)
  def _():
    pooled = sums[...] / jnp.maximum(counts[...], 1.0)
    z = jnp.maximum(
        jnp.dot(pooled, wf1_ref[...], precision=_PREC) + bf1_ref[...], 0.0)
    out_ref[...] = jnp.dot(z, wf2_ref[...], precision=_PREC) + bf2_ref[...]


def _make_tc_head(interpret=False):
  return pl.pallas_call(
      _tc_head_body,
      grid=(_GRID,),
      in_specs=[
          pl.BlockSpec((_NC, _BR, _HIDDEN), lambda i: (0, i, 0)),
          pl.BlockSpec((_BR, _HIDDEN), lambda i: (i, 0)),
          pl.BlockSpec((_BR, 1), lambda i: (i, 0)),
          pl.BlockSpec((1, _HIDDEN), lambda i: (0, 0)),
          pl.BlockSpec((_BR, 1), lambda i: (i, 0)),
          pl.BlockSpec((_HIDDEN, _HIDDEN), lambda i: (0, 0)),
          pl.BlockSpec((1, _HIDDEN), lambda i: (0, 0)),
          pl.BlockSpec((_HIDDEN, 1), lambda i: (0, 0)),
          pl.BlockSpec((1, 1), lambda i: (0, 0)),
      ],
      out_specs=pl.BlockSpec((_NUM_GRAPHS, 1), lambda i: (0, 0)),
      out_shape=jax.ShapeDtypeStruct((_NUM_GRAPHS, 1), jnp.float32),
      scratch_shapes=[
          pltpu.VMEM((_NUM_GRAPHS, _HIDDEN), jnp.float32),
          pltpu.VMEM((_NUM_GRAPHS, 1), jnp.float32),
      ],
      interpret=interpret,
  )


_sc_degree = _make_sc_degree()
_sc_aggregate = _make_sc_aggregate()
_tc_encode = _make_tc_encode()
_tc_layer = _make_tc_layer()
_tc_head = _make_tc_head()


def kernel(x, edge_index, batch, W_node, b_node, W1, b1, W2, b2, W3, b3,
           Wf1, bf1, Wf2, bf2):
  pad = _E_PAD - _N_EDGES
  src = jnp.concatenate(
      [edge_index[0].astype(jnp.int32), jnp.zeros((pad,), jnp.int32)]
  ).reshape(_NW, _CH, _K)
  dst = jnp.concatenate(
      [edge_index[1].astype(jnp.int32), jnp.full((pad,), _SINK, jnp.int32)]
  ).reshape(_NW, _CH, _K)
  zeros_deg = jnp.zeros((_STRIPE, _DEG_W), jnp.float32)
  zeros_agg = jnp.zeros((_STRIPE, _HIDDEN), jnp.float32)
  ones_deg = jnp.ones((_K, _DEG_W), jnp.float32)
  batch_f = batch.astype(jnp.float32).reshape(_N_NODES, 1)

  deg_p = _sc_degree(dst, ones_deg, zeros_deg)
  g1, dinv = _tc_encode(deg_p, x, W_node, b_node.reshape(1, _HIDDEN), W1)
  a1 = _sc_aggregate(src, dst, g1, zeros_agg)
  g2 = _tc_layer(a1, g1, dinv, b1.reshape(1, _HIDDEN), W2)
  a2 = _sc_aggregate(src, dst, g2, zeros_agg)
  g3 = _tc_layer(a2, g2, dinv, b2.reshape(1, _HIDDEN), W3)
  a3 = _sc_aggregate(src, dst, g3, zeros_agg)
  out = _tc_head(a3, g3, dinv, b3.reshape(1, _HIDDEN), batch_f,
                 Wf1, bf1.reshape(1, _HIDDEN), Wf2, bf2.reshape(1, 1))
  return out


# speedup vs baseline: 6.2729x; 6.2729x over previous
import functools

import jax
import jax.numpy as jnp
from jax import lax
from jax.experimental import pallas as pl
from jax.experimental.pallas import tpu as pltpu
from jax.experimental.pallas import tpu_sc as plsc

_N_NODES = 10000
_NODE_DIM = 128
_HIDDEN = 64
_NUM_GRAPHS = 256
_N_EDGES = 320000

_NC = 2
_NS = 16
_NW = _NC * _NS
_K = 128
_EPW = 10240
_CH = _EPW // _K
_E_PAD = _NW * _EPW
_SINK = _N_NODES
_N_ACC = 10240
_STRIPE = _N_ACC // _NS
_DEG_W = 128
_HP = 128

_BR = 1000
_GRID = _N_NODES // _BR

_PREC = lax.Precision.HIGHEST

def _mesh():
  return plsc.VectorSubcoreMesh(
      core_axis_name="c", subcore_axis_name="s",
      num_cores=_NC, num_subcores=_NS)


@functools.lru_cache(maxsize=None)
def _make_sc_degree():
  @functools.partial(
      pl.kernel,
      out_type=jax.ShapeDtypeStruct((_NC, _N_ACC, _DEG_W), jnp.float32),
      mesh=_mesh(),
      scratch_types=[
          pltpu.VMEM((_CH, _K), jnp.int32),
          pltpu.VMEM((_K, _DEG_W), jnp.float32),
          pltpu.VMEM_SHARED((_N_ACC, _DEG_W), jnp.float32),
      ],
  )
  def sc_degree(dst_hbm, ones_hbm, zeros_hbm, out_hbm, didx, ones_v, acc):
    c = lax.axis_index("c")
    s = lax.axis_index("s")
    wid = c * _NS + s
    pltpu.sync_copy(dst_hbm.at[wid], didx)
    pltpu.sync_copy(ones_hbm, ones_v)
    pltpu.sync_copy(zeros_hbm, acc.at[pl.ds(s * _STRIPE, _STRIPE)])
    plsc.subcore_barrier()

    def chunk(j, carry):
      pltpu.sync_copy(ones_v, acc.at[didx.at[j]], add=True)
      return carry

    lax.fori_loop(0, _CH, chunk, 0)
    plsc.subcore_barrier()
    pltpu.sync_copy(acc.at[pl.ds(s * _STRIPE, _STRIPE)],
                    out_hbm.at[c, pl.ds(s * _STRIPE, _STRIPE)])

  return sc_degree


@functools.lru_cache(maxsize=None)
def _make_sc_aggregate():
  @functools.partial(
      pl.kernel,
      out_type=jax.ShapeDtypeStruct((_NC, _N_ACC, _HP), jnp.float32),
      mesh=_mesh(),
      scratch_types=[
          pltpu.VMEM((_CH, _K), jnp.int32),
          pltpu.VMEM((_CH, _K), jnp.int32),
          pltpu.VMEM((_K, _HP), jnp.float32),
          pltpu.VMEM_SHARED((_N_ACC, _HP), jnp.float32),
          pltpu.SemaphoreType.DMA,
      ],
  )
  def sc_aggregate(src_hbm, dst_hbm, g_hbm, zeros_hbm, out_hbm,
                   sidx, didx, rows, acc, sem):
    c = lax.axis_index("c")
    s = lax.axis_index("s")
    wid = c * _NS + s
    pltpu.sync_copy(src_hbm.at[wid], sidx)
    pltpu.sync_copy(dst_hbm.at[wid], didx)
    pltpu.sync_copy(zeros_hbm, acc.at[pl.ds(s * _STRIPE, _STRIPE)])
    plsc.subcore_barrier()

    def chunk(j, carry):
      pltpu.async_copy(g_hbm.at[sidx.at[j]], rows, sem).wait()
      pltpu.sync_copy(rows, acc.at[didx.at[j]], add=True)
      return carry

    lax.fori_loop(0, _CH, chunk, 0)
    plsc.subcore_barrier()
    pltpu.sync_copy(acc.at[pl.ds(s * _STRIPE, _STRIPE)],
                    out_hbm.at[c, pl.ds(s * _STRIPE, _STRIPE)])

  return sc_aggregate


def _tc_encode_body(deg_ref, x_ref, wn_ref, bn_ref, w1_ref, g1_ref, dinv_ref):
  deg = deg_ref[0, :, 0:1] + deg_ref[1, :, 0:1] + 1.0
  dinv = lax.rsqrt(jnp.maximum(deg, 1.0))
  h0 = jnp.maximum(
      jnp.dot(x_ref[...], wn_ref[...], precision=_PREC) + bn_ref[...], 0.0)
  g = dinv * jnp.dot(h0, w1_ref[...], precision=_PREC)
  g1_ref[...] = jnp.concatenate([g, jnp.zeros((_BR, _HP - _HIDDEN), g.dtype)],
                                axis=1)
  dinv_ref[...] = dinv


def _make_tc_encode(interpret=False):
  return pl.pallas_call(
      _tc_encode_body,
      grid=(_GRID,),
      in_specs=[
          pl.BlockSpec((_NC, _BR, _DEG_W), lambda i: (0, i, 0)),
          pl.BlockSpec((_BR, _NODE_DIM), lambda i: (i, 0)),
          pl.BlockSpec((_NODE_DIM, _HIDDEN), lambda i: (0, 0)),
          pl.BlockSpec((1, _HIDDEN), lambda i: (0, 0)),
          pl.BlockSpec((_HIDDEN, _HIDDEN), lambda i: (0, 0)),
      ],
      out_specs=[
          pl.BlockSpec((_BR, _HP), lambda i: (i, 0)),
          pl.BlockSpec((_BR, 1), lambda i: (i, 0)),
      ],
      out_shape=[
          jax.ShapeDtypeStruct((_N_NODES, _HP), jnp.float32),
          jax.ShapeDtypeStruct((_N_NODES, 1), jnp.float32),
      ],
      interpret=interpret,
  )


def _tc_layer_body(ap_ref, g_ref, dinv_ref, b_ref, w_ref, out_ref):
  a = (ap_ref[0, :, 0:_HIDDEN] + ap_ref[1, :, 0:_HIDDEN]
       + g_ref[:, 0:_HIDDEN])
  h = jnp.maximum(dinv_ref[...] * a + b_ref[...], 0.0)
  g = dinv_ref[...] * jnp.dot(h, w_ref[...], precision=_PREC)
  out_ref[...] = jnp.concatenate([g, jnp.zeros((_BR, _HP - _HIDDEN), g.dtype)],
                                 axis=1)


def _make_tc_layer(interpret=False):
  return pl.pallas_call(
      _tc_layer_body,
      grid=(_GRID,),
      in_specs=[
          pl.BlockSpec((_NC, _BR, _HP), lambda i: (0, i, 0)),
          pl.BlockSpec((_BR, _HP), lambda i: (i, 0)),
          pl.BlockSpec((_BR, 1), lambda i: (i, 0)),
          pl.BlockSpec((1, _HIDDEN), lambda i: (0, 0)),
          pl.BlockSpec((_HIDDEN, _HIDDEN), lambda i: (0, 0)),
      ],
      out_specs=pl.BlockSpec((_BR, _HP), lambda i: (i, 0)),
      out_shape=jax.ShapeDtypeStruct((_N_NODES, _HP), jnp.float32),
      interpret=interpret,
  )


def _tc_head_body(ap_ref, g_ref, dinv_ref, b_ref, batch_ref,
                  wf1_ref, bf1_ref, wf2_ref, bf2_ref, out_ref, sums, counts):
  i = pl.program_id(0)

  @pl.when(i == 0)
  def _():
    sums[...] = jnp.zeros_like(sums)
    counts[...] = jnp.zeros_like(counts)

  a = (ap_ref[0, :, 0:_HIDDEN] + ap_ref[1, :, 0:_HIDDEN]
       + g_ref[:, 0:_HIDDEN])
  h3 = jnp.maximum(dinv_ref[...] * a + b_ref[...], 0.0)
  iota = lax.broadcasted_iota(jnp.int32, (1, _NUM_GRAPHS), 1).astype(jnp.float32)
  onehot = (batch_ref[...] == iota).astype(jnp.float32)
  dn = (((0,), (0,)), ((), ()))
  sums[...] += lax.dot_general(onehot, h3, dn, precision=_PREC)
  counts[...] += lax.dot_general(
      onehot, jnp.ones((_BR, 1), jnp.float32), dn, precision=_PREC)

  @pl.when(i == _GRID - 1)
  def _():
    pooled = sums[...] / jnp.maximum(counts[...], 1.0)
    z = jnp.maximum(
        jnp.dot(pooled, wf1_ref[...], precision=_PREC) + bf1_ref[...], 0.0)
    out_ref[...] = jnp.dot(z, wf2_ref[...], precision=_PREC) + bf2_ref[...]


def _make_tc_head(interpret=False):
  return pl.pallas_call(
      _tc_head_body,
      grid=(_GRID,),
      in_specs=[
          pl.BlockSpec((_NC, _BR, _HP), lambda i: (0, i, 0)),
          pl.BlockSpec((_BR, _HP), lambda i: (i, 0)),
          pl.BlockSpec((_BR, 1), lambda i: (i, 0)),
          pl.BlockSpec((1, _HIDDEN), lambda i: (0, 0)),
          pl.BlockSpec((_BR, 1), lambda i: (i, 0)),
          pl.BlockSpec((_HIDDEN, _HIDDEN), lambda i: (0, 0)),
          pl.BlockSpec((1, _HIDDEN), lambda i: (0, 0)),
          pl.BlockSpec((_HIDDEN, 1), lambda i: (0, 0)),
          pl.BlockSpec((1, 1), lambda i: (0, 0)),
      ],
      out_specs=pl.BlockSpec((_NUM_GRAPHS, 1), lambda i: (0, 0)),
      out_shape=jax.ShapeDtypeStruct((_NUM_GRAPHS, 1), jnp.float32),
      scratch_shapes=[
          pltpu.VMEM((_NUM_GRAPHS, _HIDDEN), jnp.float32),
          pltpu.VMEM((_NUM_GRAPHS, 1), jnp.float32),
      ],
      interpret=interpret,
  )


_tc_encode = _make_tc_encode()
_tc_layer = _make_tc_layer()
_tc_head = _make_tc_head()


def kernel(x, edge_index, batch, W_node, b_node, W1, b1, W2, b2, W3, b3,
           Wf1, bf1, Wf2, bf2):
  _sc_degree = _make_sc_degree()
  _sc_aggregate = _make_sc_aggregate()
  pad = _E_PAD - _N_EDGES
  src = jnp.concatenate(
      [edge_index[0].astype(jnp.int32), jnp.zeros((pad,), jnp.int32)]
  ).reshape(_NW, _CH, _K)
  dst = jnp.concatenate(
      [edge_index[1].astype(jnp.int32), jnp.full((pad,), _SINK, jnp.int32)]
  ).reshape(_NW, _CH, _K)
  zeros_deg = jnp.zeros((_STRIPE, _DEG_W), jnp.float32)
  zeros_agg = jnp.zeros((_STRIPE, _HP), jnp.float32)
  ones_deg = jnp.ones((_K, _DEG_W), jnp.float32)
  batch_f = batch.astype(jnp.float32).reshape(_N_NODES, 1)

  deg_p = _sc_degree(dst, ones_deg, zeros_deg)
  g1, dinv = _tc_encode(deg_p, x, W_node, b_node.reshape(1, _HIDDEN), W1)
  a1 = _sc_aggregate(src, dst, g1, zeros_agg)
  g2 = _tc_layer(a1, g1, dinv, b1.reshape(1, _HIDDEN), W2)
  a2 = _sc_aggregate(src, dst, g2, zeros_agg)
  g3 = _tc_layer(a2, g2, dinv, b2.reshape(1, _HIDDEN), W3)
  a3 = _sc_aggregate(src, dst, g3, zeros_agg)
  out = _tc_head(a3, g3, dinv, b3.reshape(1, _HIDDEN), batch_f,
                 Wf1, bf1.reshape(1, _HIDDEN), Wf2, bf2.reshape(1, 1))
  return out


# speedup vs baseline: 6.7341x; 1.0735x over previous
import functools

import jax
import jax.numpy as jnp
from jax import lax
from jax.experimental import pallas as pl
from jax.experimental.pallas import tpu as pltpu
from jax.experimental.pallas import tpu_sc as plsc

_N_NODES = 10000
_NODE_DIM = 128
_HIDDEN = 64
_NUM_GRAPHS = 256
_N_EDGES = 320000

_NC = 2
_NS = 16
_NW = _NC * _NS
_K = 128
_EPW = 10240
_CH = _EPW // _K
_E_PAD = _NW * _EPW
_SINK = _N_NODES
_N_ACC = 10240
_STRIPE = _N_ACC // _NS
_DEG_W = 128
_HP = 128
_IBC = 8

_BR = 1000
_GRID = _N_NODES // _BR

_PREC = lax.Precision.HIGHEST

def _mesh():
  return plsc.VectorSubcoreMesh(
      core_axis_name="c", subcore_axis_name="s",
      num_cores=_NC, num_subcores=_NS)


@functools.lru_cache(maxsize=None)
def _make_sc_degree():
  @functools.partial(
      pl.kernel,
      out_type=jax.ShapeDtypeStruct((_NC, _N_ACC, _DEG_W), jnp.float32),
      mesh=_mesh(),
      scratch_types=[
          pltpu.VMEM((_CH, _K), jnp.int32),
          pltpu.VMEM((_K, _DEG_W), jnp.float32),
          pltpu.VMEM_SHARED((_N_ACC, _DEG_W), jnp.float32),
      ],
  )
  def sc_degree(dst_hbm, ones_hbm, zeros_hbm, out_hbm, didx, ones_v, acc):
    c = lax.axis_index("c")
    s = lax.axis_index("s")
    wid = c * _NS + s
    pltpu.sync_copy(dst_hbm.at[wid], didx)
    pltpu.sync_copy(ones_hbm, ones_v)
    pltpu.sync_copy(zeros_hbm, acc.at[pl.ds(s * _STRIPE, _STRIPE)])
    plsc.subcore_barrier()

    def chunk(j, carry):
      pltpu.sync_copy(ones_v, acc.at[didx.at[j]], add=True)
      return carry

    lax.fori_loop(0, _CH, chunk, 0)
    plsc.subcore_barrier()
    pltpu.sync_copy(acc.at[pl.ds(s * _STRIPE, _STRIPE)],
                    out_hbm.at[c, pl.ds(s * _STRIPE, _STRIPE)])

  return sc_degree


@functools.lru_cache(maxsize=None)
def _make_sc_aggregate():
  @functools.partial(
      pl.kernel,
      out_type=jax.ShapeDtypeStruct((_NC, _N_ACC, _HP), jnp.float32),
      mesh=_mesh(),
      scratch_types=[
          pltpu.VMEM((_IBC, _K), jnp.int32),
          pltpu.VMEM((_CH, _K), jnp.int32),
          pltpu.VMEM((_K, _HP), jnp.float32),
          pltpu.VMEM((_K, _HP), jnp.float32),
          pltpu.VMEM_SHARED((_N_ACC, _HP), jnp.float32),
          pltpu.SemaphoreType.DMA,
          pltpu.SemaphoreType.DMA,
      ],
  )
  def sc_aggregate(src_hbm, dst_hbm, g_hbm, zeros_hbm, out_hbm,
                   sidx, didx, rows0, rows1, acc, sem0, sem1):
    rows = (rows0, rows1)
    sems = (sem0, sem1)
    c = lax.axis_index("c")
    s = lax.axis_index("s")
    wid = c * _NS + s
    pltpu.sync_copy(dst_hbm.at[wid], didx)
    pltpu.sync_copy(zeros_hbm, acc.at[pl.ds(s * _STRIPE, _STRIPE)])
    plsc.subcore_barrier()

    def block(i, carry):
      j0 = i * _IBC
      pltpu.sync_copy(src_hbm.at[wid, pl.ds(j0, _IBC)], sidx)
      pltpu.make_async_copy(g_hbm.at[sidx.at[0]], rows[0], sems[0]).start()
      pltpu.make_async_copy(g_hbm.at[sidx.at[1]], rows[1], sems[1]).start()
      for jj in range(_IBC):
        r = jj % 2
        pltpu.make_async_copy(g_hbm.at[sidx.at[jj]], rows[r], sems[r]).wait()
        pltpu.sync_copy(rows[r], acc.at[didx.at[j0 + jj]], add=True)
        if jj + 2 < _IBC:
          pltpu.make_async_copy(
              g_hbm.at[sidx.at[jj + 2]], rows[r], sems[r]).start()
      return carry

    lax.fori_loop(0, _CH // _IBC, block, 0)
    plsc.subcore_barrier()
    pltpu.sync_copy(acc.at[pl.ds(s * _STRIPE, _STRIPE)],
                    out_hbm.at[c, pl.ds(s * _STRIPE, _STRIPE)])

  return sc_aggregate


def _tc_encode_body(deg_ref, x_ref, wn_ref, bn_ref, w1_ref, g1_ref, dinv_ref):
  deg = deg_ref[0, :, 0:1] + deg_ref[1, :, 0:1] + 1.0
  dinv = lax.rsqrt(jnp.maximum(deg, 1.0))
  h0 = jnp.maximum(
      jnp.dot(x_ref[...], wn_ref[...], precision=_PREC) + bn_ref[...], 0.0)
  g = dinv * jnp.dot(h0, w1_ref[...], precision=_PREC)
  g1_ref[...] = jnp.concatenate([g, jnp.zeros((_BR, _HP - _HIDDEN), g.dtype)],
                                axis=1)
  dinv_ref[...] = dinv


def _make_tc_encode(interpret=False):
  return pl.pallas_call(
      _tc_encode_body,
      grid=(_GRID,),
      in_specs=[
          pl.BlockSpec((_NC, _BR, _DEG_W), lambda i: (0, i, 0)),
          pl.BlockSpec((_BR, _NODE_DIM), lambda i: (i, 0)),
          pl.BlockSpec((_NODE_DIM, _HIDDEN), lambda i: (0, 0)),
          pl.BlockSpec((1, _HIDDEN), lambda i: (0, 0)),
          pl.BlockSpec((_HIDDEN, _HIDDEN), lambda i: (0, 0)),
      ],
      out_specs=[
          pl.BlockSpec((_BR, _HP), lambda i: (i, 0)),
          pl.BlockSpec((_BR, 1), lambda i: (i, 0)),
      ],
      out_shape=[
          jax.ShapeDtypeStruct((_N_NODES, _HP), jnp.float32),
          jax.ShapeDtypeStruct((_N_NODES, 1), jnp.float32),
      ],
      interpret=interpret,
  )


def _tc_layer_body(ap_ref, g_ref, dinv_ref, b_ref, w_ref, out_ref):
  a = (ap_ref[0, :, 0:_HIDDEN] + ap_ref[1, :, 0:_HIDDEN]
       + g_ref[:, 0:_HIDDEN])
  h = jnp.maximum(dinv_ref[...] * a + b_ref[...], 0.0)
  g = dinv_ref[...] * jnp.dot(h, w_ref[...], precision=_PREC)
  out_ref[...] = jnp.concatenate([g, jnp.zeros((_BR, _HP - _HIDDEN), g.dtype)],
                                 axis=1)


def _make_tc_layer(interpret=False):
  return pl.pallas_call(
      _tc_layer_body,
      grid=(_GRID,),
      in_specs=[
          pl.BlockSpec((_NC, _BR, _HP), lambda i: (0, i, 0)),
          pl.BlockSpec((_BR, _HP), lambda i: (i, 0)),
          pl.BlockSpec((_BR, 1), lambda i: (i, 0)),
          pl.BlockSpec((1, _HIDDEN), lambda i: (0, 0)),
          pl.BlockSpec((_HIDDEN, _HIDDEN), lambda i: (0, 0)),
      ],
      out_specs=pl.BlockSpec((_BR, _HP), lambda i: (i, 0)),
      out_shape=jax.ShapeDtypeStruct((_N_NODES, _HP), jnp.float32),
      interpret=interpret,
  )


def _tc_head_body(ap_ref, g_ref, dinv_ref, b_ref, batch_ref,
                  wf1_ref, bf1_ref, wf2_ref, bf2_ref, out_ref, sums, counts):
  i = pl.program_id(0)

  @pl.when(i == 0)
  def _():
    sums[...] = jnp.zeros_like(sums)
    counts[...] = jnp.zeros_like(counts)

  a = (ap_ref[0, :, 0:_HIDDEN] + ap_ref[1, :, 0:_HIDDEN]
       + g_ref[:, 0:_HIDDEN])
  h3 = jnp.maximum(dinv_ref[...] * a + b_ref[...], 0.0)
  iota = lax.broadcasted_iota(jnp.int32, (1, _NUM_GRAPHS), 1).astype(jnp.float32)
  onehot = (batch_ref[...] == iota).astype(jnp.float32)
  dn = (((0,), (0,)), ((), ()))
  sums[...] += lax.dot_general(onehot, h3, dn, precision=_PREC)
  counts[...] += lax.dot_general(
      onehot, jnp.ones((_BR, 1), jnp.float32), dn, precision=_PREC)

  @pl.when(i == _GRID - 1)
  def _():
    pooled = sums[...] / jnp.maximum(counts[...], 1.0)
    z = jnp.maximum(
        jnp.dot(pooled, wf1_ref[...], precision=_PREC) + bf1_ref[...], 0.0)
    out_ref[...] = jnp.dot(z, wf2_ref[...], precision=_PREC) + bf2_ref[...]


def _make_tc_head(interpret=False):
  return pl.pallas_call(
      _tc_head_body,
      grid=(_GRID,),
      in_specs=[
          pl.BlockSpec((_NC, _BR, _HP), lambda i: (0, i, 0)),
          pl.BlockSpec((_BR, _HP), lambda i: (i, 0)),
          pl.BlockSpec((_BR, 1), lambda i: (i, 0)),
          pl.BlockSpec((1, _HIDDEN), lambda i: (0, 0)),
          pl.BlockSpec((_BR, 1), lambda i: (i, 0)),
          pl.BlockSpec((_HIDDEN, _HIDDEN), lambda i: (0, 0)),
          pl.BlockSpec((1, _HIDDEN), lambda i: (0, 0)),
          pl.BlockSpec((_HIDDEN, 1), lambda i: (0, 0)),
          pl.BlockSpec((1, 1), lambda i: (0, 0)),
      ],
      out_specs=pl.BlockSpec((_NUM_GRAPHS, 1), lambda i: (0, 0)),
      out_shape=jax.ShapeDtypeStruct((_NUM_GRAPHS, 1), jnp.float32),
      scratch_shapes=[
          pltpu.VMEM((_NUM_GRAPHS, _HIDDEN), jnp.float32),
          pltpu.VMEM((_NUM_GRAPHS, 1), jnp.float32),
      ],
      interpret=interpret,
  )


_tc_encode = _make_tc_encode()
_tc_layer = _make_tc_layer()
_tc_head = _make_tc_head()


def kernel(x, edge_index, batch, W_node, b_node, W1, b1, W2, b2, W3, b3,
           Wf1, bf1, Wf2, bf2):
  _sc_degree = _make_sc_degree()
  _sc_aggregate = _make_sc_aggregate()
  pad = _E_PAD - _N_EDGES
  src = jnp.concatenate(
      [edge_index[0].astype(jnp.int32), jnp.zeros((pad,), jnp.int32)]
  ).reshape(_NW, _CH, _K)
  dst = jnp.concatenate(
      [edge_index[1].astype(jnp.int32), jnp.full((pad,), _SINK, jnp.int32)]
  ).reshape(_NW, _CH, _K)
  zeros_deg = jnp.zeros((_STRIPE, _DEG_W), jnp.float32)
  zeros_agg = jnp.zeros((_STRIPE, _HP), jnp.float32)
  ones_deg = jnp.ones((_K, _DEG_W), jnp.float32)
  batch_f = batch.astype(jnp.float32).reshape(_N_NODES, 1)

  deg_p = _sc_degree(dst, ones_deg, zeros_deg)
  g1, dinv = _tc_encode(deg_p, x, W_node, b_node.reshape(1, _HIDDEN), W1)
  a1 = _sc_aggregate(src, dst, g1, zeros_agg)
  g2 = _tc_layer(a1, g1, dinv, b1.reshape(1, _HIDDEN), W2)
  a2 = _sc_aggregate(src, dst, g2, zeros_agg)
  g3 = _tc_layer(a2, g2, dinv, b2.reshape(1, _HIDDEN), W3)
  a3 = _sc_aggregate(src, dst, g3, zeros_agg)
  out = _tc_head(a3, g3, dinv, b3.reshape(1, _HIDDEN), batch_f,
                 Wf1, bf1.reshape(1, _HIDDEN), Wf2, bf2.reshape(1, 1))
  return out


# speedup vs baseline: 20.7626x; 3.0832x over previous
import functools

import jax
import jax.numpy as jnp
from jax import lax
from jax.experimental import pallas as pl
from jax.experimental.pallas import tpu as pltpu
from jax.experimental.pallas import tpu_sc as plsc

_N_NODES = 10000
_NODE_DIM = 128
_HIDDEN = 64
_NUM_GRAPHS = 256
_N_EDGES = 320000

_NC = 2
_NS = 16
_NW = _NC * _NS
_K = 128
_EPW = 10240
_CH = _EPW // _K
_E_PAD = _NW * _EPW
_SINK = _N_NODES
_N_ACC = 10240
_STRIPE = _N_ACC // _NS
_DEG_W = 128
_HP = 128
_IBC = 8

_BR = 1000
_GRID = _N_NODES // _BR

_PREC = lax.Precision.HIGHEST

def _mesh():
  return plsc.VectorSubcoreMesh(
      core_axis_name="c", subcore_axis_name="s",
      num_cores=_NC, num_subcores=_NS)


@functools.lru_cache(maxsize=None)
def _make_sc_degree():
  @functools.partial(
      pl.kernel,
      out_type=jax.ShapeDtypeStruct((_NC, _N_ACC, _DEG_W), jnp.float32),
      mesh=_mesh(),
      scratch_types=[
          pltpu.VMEM((_CH, _K), jnp.int32),
          pltpu.VMEM((_K, _DEG_W), jnp.float32),
          pltpu.VMEM_SHARED((_N_ACC, _DEG_W), jnp.float32),
      ],
  )
  def sc_degree(dst_hbm, ones_hbm, zeros_hbm, out_hbm, didx, ones_v, acc):
    c = lax.axis_index("c")
    s = lax.axis_index("s")
    wid = c * _NS + s
    pltpu.sync_copy(dst_hbm.at[wid], didx)
    pltpu.sync_copy(ones_hbm, ones_v)
    pltpu.sync_copy(zeros_hbm, acc.at[pl.ds(s * _STRIPE, _STRIPE)])
    plsc.subcore_barrier()

    def chunk(j, carry):
      pltpu.sync_copy(ones_v, acc.at[didx.at[j]], add=True)
      return carry

    lax.fori_loop(0, _CH, chunk, 0)
    plsc.subcore_barrier()
    pltpu.sync_copy(acc.at[pl.ds(s * _STRIPE, _STRIPE)],
                    out_hbm.at[c, pl.ds(s * _STRIPE, _STRIPE)])

  return sc_degree


@functools.lru_cache(maxsize=None)
def _make_sc_aggregate():
  @functools.partial(
      pl.kernel,
      out_type=jax.ShapeDtypeStruct((_NC, _N_ACC, _HP), jnp.float32),
      mesh=_mesh(),
      scratch_types=[
          pltpu.VMEM((_IBC, _K), jnp.int32),
          pltpu.VMEM((_CH, _K), jnp.int32),
          pltpu.VMEM((_K, _HP), jnp.float32),
          pltpu.VMEM((_K, _HP), jnp.float32),
          pltpu.VMEM_SHARED((_N_ACC, _HP), jnp.float32),
          pltpu.SemaphoreType.DMA,
          pltpu.SemaphoreType.DMA,
      ],
  )
  def sc_aggregate(src_hbm, dst_hbm, g_hbm, zeros_hbm, out_hbm,
                   sidx, didx, rows0, rows1, acc, sem0, sem1):
    rows = (rows0, rows1)
    sems = (sem0, sem1)
    c = lax.axis_index("c")
    s = lax.axis_index("s")
    wid = c * _NS + s
    pltpu.sync_copy(dst_hbm.at[wid], didx)
    pltpu.sync_copy(zeros_hbm, acc.at[pl.ds(s * _STRIPE, _STRIPE)])
    plsc.subcore_barrier()

    def block(i, carry):
      j0 = i * _IBC
      pltpu.sync_copy(src_hbm.at[wid, pl.ds(j0, _IBC)], sidx)
      pltpu.make_async_copy(g_hbm.at[sidx.at[0]], rows[0], sems[0]).start()
      pltpu.make_async_copy(g_hbm.at[sidx.at[1]], rows[1], sems[1]).start()
      for jj in range(_IBC):
        r = jj % 2
        pltpu.make_async_copy(g_hbm.at[sidx.at[jj]], rows[r], sems[r]).wait()
        pltpu.sync_copy(rows[r], acc.at[didx.at[j0 + jj]], add=True)
        if jj + 2 < _IBC:
          pltpu.make_async_copy(
              g_hbm.at[sidx.at[jj + 2]], rows[r], sems[r]).start()
      return carry

    lax.fori_loop(0, _CH // _IBC, block, 0)
    plsc.subcore_barrier()
    pltpu.sync_copy(acc.at[pl.ds(s * _STRIPE, _STRIPE)],
                    out_hbm.at[c, pl.ds(s * _STRIPE, _STRIPE)])

  return sc_aggregate


def _tc_encode_body(deg_ref, x_ref, wn_ref, bn_ref, w1_ref, g1_ref, dinv_ref):
  deg = deg_ref[0, :, 0:1] + deg_ref[1, :, 0:1] + 1.0
  dinv = lax.rsqrt(jnp.maximum(deg, 1.0))
  h0 = jnp.maximum(
      jnp.dot(x_ref[...], wn_ref[...], precision=_PREC) + bn_ref[...], 0.0)
  g = dinv * jnp.dot(h0, w1_ref[...], precision=_PREC)
  g1_ref[...] = jnp.concatenate([g, jnp.zeros((_BR, _HP - _HIDDEN), g.dtype)],
                                axis=1)
  dinv_ref[...] = dinv


def _make_tc_encode(interpret=False):
  return pl.pallas_call(
      _tc_encode_body,
      grid=(_GRID,),
      in_specs=[
          pl.BlockSpec((_NC, _BR, _DEG_W), lambda i: (0, i, 0)),
          pl.BlockSpec((_BR, _NODE_DIM), lambda i: (i, 0)),
          pl.BlockSpec((_NODE_DIM, _HIDDEN), lambda i: (0, 0)),
          pl.BlockSpec((1, _HIDDEN), lambda i: (0, 0)),
          pl.BlockSpec((_HIDDEN, _HIDDEN), lambda i: (0, 0)),
      ],
      out_specs=[
          pl.BlockSpec((_BR, _HP), lambda i: (i, 0)),
          pl.BlockSpec((_BR, 1), lambda i: (i, 0)),
      ],
      out_shape=[
          jax.ShapeDtypeStruct((_N_NODES, _HP), jnp.float32),
          jax.ShapeDtypeStruct((_N_NODES, 1), jnp.float32),
      ],
      interpret=interpret,
  )


def _tc_layer_body(ap_ref, g_ref, dinv_ref, b_ref, w_ref, out_ref):
  a = (ap_ref[0, :, 0:_HIDDEN] + ap_ref[1, :, 0:_HIDDEN]
       + g_ref[:, 0:_HIDDEN])
  h = jnp.maximum(dinv_ref[...] * a + b_ref[...], 0.0)
  g = dinv_ref[...] * jnp.dot(h, w_ref[...], precision=_PREC)
  out_ref[...] = jnp.concatenate([g, jnp.zeros((_BR, _HP - _HIDDEN), g.dtype)],
                                 axis=1)


def _make_tc_layer(interpret=False):
  return pl.pallas_call(
      _tc_layer_body,
      grid=(_GRID,),
      in_specs=[
          pl.BlockSpec((_NC, _BR, _HP), lambda i: (0, i, 0)),
          pl.BlockSpec((_BR, _HP), lambda i: (i, 0)),
          pl.BlockSpec((_BR, 1), lambda i: (i, 0)),
          pl.BlockSpec((1, _HIDDEN), lambda i: (0, 0)),
          pl.BlockSpec((_HIDDEN, _HIDDEN), lambda i: (0, 0)),
      ],
      out_specs=pl.BlockSpec((_BR, _HP), lambda i: (i, 0)),
      out_shape=jax.ShapeDtypeStruct((_N_NODES, _HP), jnp.float32),
      interpret=interpret,
  )


def _tc_head_body(ap_ref, g_ref, dinv_ref, b_ref, batch_ref,
                  wf1_ref, bf1_ref, wf2_ref, bf2_ref, out_ref, sums, counts):
  i = pl.program_id(0)

  @pl.when(i == 0)
  def _():
    sums[...] = jnp.zeros_like(sums)
    counts[...] = jnp.zeros_like(counts)

  a = (ap_ref[0, :, 0:_HIDDEN] + ap_ref[1, :, 0:_HIDDEN]
       + g_ref[:, 0:_HIDDEN])
  h3 = jnp.maximum(dinv_ref[...] * a + b_ref[...], 0.0)
  iota = lax.broadcasted_iota(jnp.int32, (1, _NUM_GRAPHS), 1).astype(jnp.float32)
  onehot = (batch_ref[...] == iota).astype(jnp.float32)
  dn = (((0,), (0,)), ((), ()))
  sums[...] += lax.dot_general(onehot, h3, dn, precision=_PREC)
  counts[...] += lax.dot_general(
      onehot, jnp.ones((_BR, 1), jnp.float32), dn, precision=_PREC)

  @pl.when(i == _GRID - 1)
  def _():
    pooled = sums[...] / jnp.maximum(counts[...], 1.0)
    z = jnp.maximum(
        jnp.dot(pooled, wf1_ref[...], precision=_PREC) + bf1_ref[...], 0.0)
    out_ref[...] = jnp.dot(z, wf2_ref[...], precision=_PREC) + bf2_ref[...]


def _make_tc_head(interpret=False):
  return pl.pallas_call(
      _tc_head_body,
      grid=(_GRID,),
      in_specs=[
          pl.BlockSpec((_NC, _BR, _HP), lambda i: (0, i, 0)),
          pl.BlockSpec((_BR, _HP), lambda i: (i, 0)),
          pl.BlockSpec((_BR, 1), lambda i: (i, 0)),
          pl.BlockSpec((1, _HIDDEN), lambda i: (0, 0)),
          pl.BlockSpec((_BR, 1), lambda i: (i, 0)),
          pl.BlockSpec((_HIDDEN, _HIDDEN), lambda i: (0, 0)),
          pl.BlockSpec((1, _HIDDEN), lambda i: (0, 0)),
          pl.BlockSpec((_HIDDEN, 1), lambda i: (0, 0)),
          pl.BlockSpec((1, 1), lambda i: (0, 0)),
      ],
      out_specs=pl.BlockSpec((_NUM_GRAPHS, 1), lambda i: (0, 0)),
      out_shape=jax.ShapeDtypeStruct((_NUM_GRAPHS, 1), jnp.float32),
      scratch_shapes=[
          pltpu.VMEM((_NUM_GRAPHS, _HIDDEN), jnp.float32),
          pltpu.VMEM((_NUM_GRAPHS, 1), jnp.float32),
      ],
      interpret=interpret,
  )


_tc_encode = _make_tc_encode()
_tc_layer = _make_tc_layer()
_tc_head = _make_tc_head()


def kernel(x, edge_index, batch, W_node, b_node, W1, b1, W2, b2, W3, b3,
           Wf1, bf1, Wf2, bf2):
  _sc_degree = _make_sc_degree()
  _sc_aggregate = _make_sc_aggregate()
  pad = _E_PAD - _N_EDGES
  pad_i = jnp.arange(pad, dtype=jnp.int32)
  src = jnp.concatenate(
      [edge_index[0].astype(jnp.int32), pad_i % _N_NODES]
  ).reshape(_NW, _CH, _K)
  dst = jnp.concatenate(
      [edge_index[1].astype(jnp.int32),
       _SINK + pad_i % (_N_ACC - _N_NODES)]
  ).reshape(_NW, _CH, _K)
  zeros_deg = jnp.zeros((_STRIPE, _DEG_W), jnp.float32)
  zeros_agg = jnp.zeros((_STRIPE, _HP), jnp.float32)
  ones_deg = jnp.ones((_K, _DEG_W), jnp.float32)
  batch_f = batch.astype(jnp.float32).reshape(_N_NODES, 1)

  deg_p = _sc_degree(dst, ones_deg, zeros_deg)
  g1, dinv = _tc_encode(deg_p, x, W_node, b_node.reshape(1, _HIDDEN), W1)
  a1 = _sc_aggregate(src, dst, g1, zeros_agg)
  g2 = _tc_layer(a1, g1, dinv, b1.reshape(1, _HIDDEN), W2)
  a2 = _sc_aggregate(src, dst, g2, zeros_agg)
  g3 = _tc_layer(a2, g2, dinv, b2.reshape(1, _HIDDEN), W3)
  a3 = _sc_aggregate(src, dst, g3, zeros_agg)
  out = _tc_head(a3, g3, dinv, b3.reshape(1, _HIDDEN), batch_f,
                 Wf1, bf1.reshape(1, _HIDDEN), Wf2, bf2.reshape(1, 1))
  return out


# speedup vs baseline: 21.6300x; 1.0418x over previous
import functools

import jax
import jax.numpy as jnp
from jax import lax
from jax.experimental import pallas as pl
from jax.experimental.pallas import tpu as pltpu
from jax.experimental.pallas import tpu_sc as plsc

_N_NODES = 10000
_NODE_DIM = 128
_HIDDEN = 64
_NUM_GRAPHS = 256
_N_EDGES = 320000

_NC = 2
_NS = 16
_NW = _NC * _NS
_K = 128
_EPW = 10240
_CH = _EPW // _K
_E_PAD = _NW * _EPW
_SINK = _N_NODES
_N_ACC = 10240
_STRIPE = _N_ACC // _NS
_DEG_W = 128
_HP = 128
_IBC = 8

_BR = 1000
_GRID = _N_NODES // _BR

_PREC = lax.Precision.HIGHEST

def _mesh():
  return plsc.VectorSubcoreMesh(
      core_axis_name="c", subcore_axis_name="s",
      num_cores=_NC, num_subcores=_NS)


@functools.lru_cache(maxsize=None)
def _make_sc_degree():
  @functools.partial(
      pl.kernel,
      out_type=jax.ShapeDtypeStruct((_NC, _N_ACC, _DEG_W), jnp.float32),
      mesh=_mesh(),
      scratch_types=[
          pltpu.VMEM((_CH, _K), jnp.int32),
          pltpu.VMEM((_K, _DEG_W), jnp.float32),
          pltpu.VMEM_SHARED((_N_ACC, _DEG_W), jnp.float32),
      ],
  )
  def sc_degree(dst_hbm, ones_hbm, zeros_hbm, out_hbm, didx, ones_v, acc):
    c = lax.axis_index("c")
    s = lax.axis_index("s")
    wid = c * _NS + s
    pltpu.sync_copy(dst_hbm.at[wid], didx)
    pltpu.sync_copy(ones_hbm, ones_v)
    pltpu.sync_copy(zeros_hbm, acc.at[pl.ds(s * _STRIPE, _STRIPE)])
    plsc.subcore_barrier()

    def chunk(j, carry):
      pltpu.sync_copy(ones_v, acc.at[didx.at[j]], add=True)
      return carry

    lax.fori_loop(0, _CH, chunk, 0)
    plsc.subcore_barrier()
    pltpu.sync_copy(acc.at[pl.ds(s * _STRIPE, _STRIPE)],
                    out_hbm.at[c, pl.ds(s * _STRIPE, _STRIPE)])

  return sc_degree


@functools.lru_cache(maxsize=None)
def _make_sc_aggregate():
  @functools.partial(
      pl.kernel,
      out_type=jax.ShapeDtypeStruct((_NC, _N_ACC, _HP), jnp.float32),
      mesh=_mesh(),
      scratch_types=[
          pltpu.VMEM((_IBC, _K), jnp.int32),
          pltpu.VMEM((_CH, _K), jnp.int32),
          pltpu.VMEM((_K, _HP), jnp.float32),
          pltpu.VMEM((_K, _HP), jnp.float32),
          pltpu.VMEM_SHARED((_N_ACC, _HP), jnp.float32),
          pltpu.SemaphoreType.DMA,
          pltpu.SemaphoreType.DMA,
      ],
  )
  def sc_aggregate(src_hbm, dst_hbm, g_hbm, zeros_hbm, out_hbm,
                   sidx, didx, rows0, rows1, acc, sem0, sem1):
    rows = (rows0, rows1)
    sems = (sem0, sem1)
    c = lax.axis_index("c")
    s = lax.axis_index("s")
    wid = c * _NS + s
    pltpu.sync_copy(dst_hbm.at[wid], didx)
    pltpu.sync_copy(zeros_hbm, acc.at[pl.ds(s * _STRIPE, _STRIPE)])
    plsc.subcore_barrier()

    def block(i, carry):
      j0 = i * _IBC
      pltpu.sync_copy(src_hbm.at[wid, pl.ds(j0, _IBC)], sidx)
      pltpu.make_async_copy(g_hbm.at[sidx.at[0]], rows[0], sems[0]).start()
      pltpu.make_async_copy(g_hbm.at[sidx.at[1]], rows[1], sems[1]).start()
      for jj in range(_IBC):
        r = jj % 2
        pltpu.make_async_copy(g_hbm.at[sidx.at[jj]], rows[r], sems[r]).wait()
        pltpu.sync_copy(rows[r], acc.at[didx.at[j0 + jj]], add=True)
        if jj + 2 < _IBC:
          pltpu.make_async_copy(
              g_hbm.at[sidx.at[jj + 2]], rows[r], sems[r]).start()
      return carry

    lax.fori_loop(0, _CH // _IBC, block, 0)
    plsc.subcore_barrier()
    pltpu.sync_copy(acc.at[pl.ds(s * _STRIPE, _STRIPE)],
                    out_hbm.at[c, pl.ds(s * _STRIPE, _STRIPE)])

  return sc_aggregate


def _tc_encode0_body(x_ref, wn_ref, bn_ref, w1_ref, u1_ref):
  h0 = jnp.maximum(
      jnp.dot(x_ref[...], wn_ref[...], precision=_PREC) + bn_ref[...], 0.0)
  u1_ref[...] = jnp.dot(h0, w1_ref[...], precision=_PREC)


def _make_tc_encode0(interpret=False):
  return pl.pallas_call(
      _tc_encode0_body,
      grid=(_GRID,),
      in_specs=[
          pl.BlockSpec((_BR, _NODE_DIM), lambda i: (i, 0)),
          pl.BlockSpec((_NODE_DIM, _HIDDEN), lambda i: (0, 0)),
          pl.BlockSpec((1, _HIDDEN), lambda i: (0, 0)),
          pl.BlockSpec((_HIDDEN, _HIDDEN), lambda i: (0, 0)),
      ],
      out_specs=pl.BlockSpec((_BR, _HIDDEN), lambda i: (i, 0)),
      out_shape=jax.ShapeDtypeStruct((_N_NODES, _HIDDEN), jnp.float32),
      interpret=interpret,
  )


def _tc_scale_body(deg_ref, u1_ref, g1_ref, dinv_ref):
  deg = deg_ref[0, :, 0:1] + deg_ref[1, :, 0:1] + 1.0
  dinv = lax.rsqrt(jnp.maximum(deg, 1.0))
  g = dinv * u1_ref[...]
  g1_ref[...] = jnp.concatenate([g, jnp.zeros((_BR, _HP - _HIDDEN), g.dtype)],
                                axis=1)
  dinv_ref[...] = dinv


def _make_tc_scale(interpret=False):
  return pl.pallas_call(
      _tc_scale_body,
      grid=(_GRID,),
      in_specs=[
          pl.BlockSpec((_NC, _BR, _DEG_W), lambda i: (0, i, 0)),
          pl.BlockSpec((_BR, _HIDDEN), lambda i: (i, 0)),
      ],
      out_specs=[
          pl.BlockSpec((_BR, _HP), lambda i: (i, 0)),
          pl.BlockSpec((_BR, 1), lambda i: (i, 0)),
      ],
      out_shape=[
          jax.ShapeDtypeStruct((_N_NODES, _HP), jnp.float32),
          jax.ShapeDtypeStruct((_N_NODES, 1), jnp.float32),
      ],
      interpret=interpret,
  )


def _tc_layer_body(ap_ref, g_ref, dinv_ref, b_ref, w_ref, out_ref):
  a = (ap_ref[0, :, 0:_HIDDEN] + ap_ref[1, :, 0:_HIDDEN]
       + g_ref[:, 0:_HIDDEN])
  h = jnp.maximum(dinv_ref[...] * a + b_ref[...], 0.0)
  g = dinv_ref[...] * jnp.dot(h, w_ref[...], precision=_PREC)
  out_ref[...] = jnp.concatenate([g, jnp.zeros((_BR, _HP - _HIDDEN), g.dtype)],
                                 axis=1)


def _make_tc_layer(interpret=False):
  return pl.pallas_call(
      _tc_layer_body,
      grid=(_GRID,),
      in_specs=[
          pl.BlockSpec((_NC, _BR, _HP), lambda i: (0, i, 0)),
          pl.BlockSpec((_BR, _HP), lambda i: (i, 0)),
          pl.BlockSpec((_BR, 1), lambda i: (i, 0)),
          pl.BlockSpec((1, _HIDDEN), lambda i: (0, 0)),
          pl.BlockSpec((_HIDDEN, _HIDDEN), lambda i: (0, 0)),
      ],
      out_specs=pl.BlockSpec((_BR, _HP), lambda i: (i, 0)),
      out_shape=jax.ShapeDtypeStruct((_N_NODES, _HP), jnp.float32),
      interpret=interpret,
  )


def _tc_head_body(ap_ref, g_ref, dinv_ref, b_ref, batch_ref,
                  wf1_ref, bf1_ref, wf2_ref, bf2_ref, out_ref, sums, counts):
  i = pl.program_id(0)

  @pl.when(i == 0)
  def _():
    sums[...] = jnp.zeros_like(sums)
    counts[...] = jnp.zeros_like(counts)

  a = (ap_ref[0, :, 0:_HIDDEN] + ap_ref[1, :, 0:_HIDDEN]
       + g_ref[:, 0:_HIDDEN])
  h3 = jnp.maximum(dinv_ref[...] * a + b_ref[...], 0.0)
  iota = lax.broadcasted_iota(jnp.int32, (1, _NUM_GRAPHS), 1).astype(jnp.float32)
  onehot = (batch_ref[...] == iota).astype(jnp.float32)
  dn = (((0,), (0,)), ((), ()))
  sums[...] += lax.dot_general(onehot, h3, dn, precision=_PREC)
  counts[...] += lax.dot_general(
      onehot, jnp.ones((_BR, 1), jnp.float32), dn, precision=_PREC)

  @pl.when(i == _GRID - 1)
  def _():
    pooled = sums[...] / jnp.maximum(counts[...], 1.0)
    z = jnp.maximum(
        jnp.dot(pooled, wf1_ref[...], precision=_PREC) + bf1_ref[...], 0.0)
    out_ref[...] = jnp.dot(z, wf2_ref[...], precision=_PREC) + bf2_ref[...]


def _make_tc_head(interpret=False):
  return pl.pallas_call(
      _tc_head_body,
      grid=(_GRID,),
      in_specs=[
          pl.BlockSpec((_NC, _BR, _HP), lambda i: (0, i, 0)),
          pl.BlockSpec((_BR, _HP), lambda i: (i, 0)),
          pl.BlockSpec((_BR, 1), lambda i: (i, 0)),
          pl.BlockSpec((1, _HIDDEN), lambda i: (0, 0)),
          pl.BlockSpec((_BR, 1), lambda i: (i, 0)),
          pl.BlockSpec((_HIDDEN, _HIDDEN), lambda i: (0, 0)),
          pl.BlockSpec((1, _HIDDEN), lambda i: (0, 0)),
          pl.BlockSpec((_HIDDEN, 1), lambda i: (0, 0)),
          pl.BlockSpec((1, 1), lambda i: (0, 0)),
      ],
      out_specs=pl.BlockSpec((_NUM_GRAPHS, 1), lambda i: (0, 0)),
      out_shape=jax.ShapeDtypeStruct((_NUM_GRAPHS, 1), jnp.float32),
      scratch_shapes=[
          pltpu.VMEM((_NUM_GRAPHS, _HIDDEN), jnp.float32),
          pltpu.VMEM((_NUM_GRAPHS, 1), jnp.float32),
      ],
      interpret=interpret,
  )


_tc_encode0 = _make_tc_encode0()
_tc_scale = _make_tc_scale()
_tc_layer = _make_tc_layer()
_tc_head = _make_tc_head()


def kernel(x, edge_index, batch, W_node, b_node, W1, b1, W2, b2, W3, b3,
           Wf1, bf1, Wf2, bf2):
  _sc_degree = _make_sc_degree()
  _sc_aggregate = _make_sc_aggregate()
  pad = _E_PAD - _N_EDGES
  pad_i = jnp.arange(pad, dtype=jnp.int32)
  src = jnp.concatenate(
      [edge_index[0].astype(jnp.int32), pad_i % _N_NODES]
  ).reshape(_NW, _CH, _K)
  dst = jnp.concatenate(
      [edge_index[1].astype(jnp.int32),
       _SINK + pad_i % (_N_ACC - _N_NODES)]
  ).reshape(_NW, _CH, _K)
  zeros_deg = jnp.zeros((_STRIPE, _DEG_W), jnp.float32)
  zeros_agg = jnp.zeros((_STRIPE, _HP), jnp.float32)
  ones_deg = jnp.ones((_K, _DEG_W), jnp.float32)
  batch_f = batch.astype(jnp.float32).reshape(_N_NODES, 1)

  deg_p = _sc_degree(dst, ones_deg, zeros_deg)
  u1 = _tc_encode0(x, W_node, b_node.reshape(1, _HIDDEN), W1)
  g1, dinv = _tc_scale(deg_p, u1)
  a1 = _sc_aggregate(src, dst, g1, zeros_agg)
  g2 = _tc_layer(a1, g1, dinv, b1.reshape(1, _HIDDEN), W2)
  a2 = _sc_aggregate(src, dst, g2, zeros_agg)
  g3 = _tc_layer(a2, g2, dinv, b2.reshape(1, _HIDDEN), W3)
  a3 = _sc_aggregate(src, dst, g3, zeros_agg)
  out = _tc_head(a3, g3, dinv, b3.reshape(1, _HIDDEN), batch_f,
                 Wf1, bf1.reshape(1, _HIDDEN), Wf2, bf2.reshape(1, 1))
  return out


# speedup vs baseline: 21.9913x; 1.0167x over previous
import functools

import jax
import jax.numpy as jnp
from jax import lax
from jax.experimental import pallas as pl
from jax.experimental.pallas import tpu as pltpu
from jax.experimental.pallas import tpu_sc as plsc

_N_NODES = 10000
_NODE_DIM = 128
_HIDDEN = 64
_NUM_GRAPHS = 256
_N_EDGES = 320000

_NC = 2
_NS = 16
_NW = _NC * _NS
_K = 128
_EPW = 10240
_CH = _EPW // _K
_E_PAD = _NW * _EPW
_SINK = _N_NODES
_N_ACC = 10240
_STRIPE = _N_ACC // _NS
_DEG_W = 128
_HP = 128
_IBC = 8

_BR = 2000
_GRID = _N_NODES // _BR

_PREC = lax.Precision.HIGHEST

def _mesh():
  return plsc.VectorSubcoreMesh(
      core_axis_name="c", subcore_axis_name="s",
      num_cores=_NC, num_subcores=_NS)


@functools.lru_cache(maxsize=None)
def _make_sc_degree():
  @functools.partial(
      pl.kernel,
      out_type=jax.ShapeDtypeStruct((_NC, _N_ACC, _DEG_W), jnp.float32),
      mesh=_mesh(),
      scratch_types=[
          pltpu.VMEM((_CH, _K), jnp.int32),
          pltpu.VMEM((_K, _DEG_W), jnp.float32),
          pltpu.VMEM_SHARED((_N_ACC, _DEG_W), jnp.float32),
      ],
  )
  def sc_degree(dst_hbm, ones_hbm, zeros_hbm, out_hbm, didx, ones, acc):
    c = lax.axis_index("c")
    s = lax.axis_index("s")
    wid = c * _NS + s
    pltpu.sync_copy(dst_hbm.at[wid], didx)
    pltpu.sync_copy(ones_hbm, ones)
    pltpu.sync_copy(zeros_hbm, acc.at[pl.ds(s * _STRIPE, _STRIPE)])
    plsc.subcore_barrier()

    def step(j, carry):
      pltpu.sync_copy(ones, acc.at[didx.at[j]], add=True)
      return carry

    lax.fori_loop(0, _CH, step, 0)
    plsc.subcore_barrier()
    pltpu.sync_copy(acc.at[pl.ds(s * _STRIPE, _STRIPE)],
                    out_hbm.at[c, pl.ds(s * _STRIPE, _STRIPE)])

  return sc_degree


@functools.lru_cache(maxsize=None)
def _make_sc_aggregate():
  @functools.partial(
      pl.kernel,
      out_type=jax.ShapeDtypeStruct((_NC, _N_ACC, _HP), jnp.float32),
      mesh=_mesh(),
      scratch_types=[
          pltpu.VMEM((_IBC, _K), jnp.int32),
          pltpu.VMEM((_CH, _K), jnp.int32),
          pltpu.VMEM((_K, _HP), jnp.float32),
          pltpu.VMEM((_K, _HP), jnp.float32),
          pltpu.VMEM_SHARED((_N_ACC, _HP), jnp.float32),
          pltpu.SemaphoreType.DMA,
          pltpu.SemaphoreType.DMA,
      ],
  )
  def sc_aggregate(src_hbm, dst_hbm, g_hbm, zeros_hbm, out_hbm,
                   sidx, didx, rows0, rows1, acc, sem0, sem1):
    rows = (rows0, rows1)
    sems = (sem0, sem1)
    c = lax.axis_index("c")
    s = lax.axis_index("s")
    wid = c * _NS + s
    pltpu.sync_copy(dst_hbm.at[wid], didx)
    pltpu.sync_copy(zeros_hbm, acc.at[pl.ds(s * _STRIPE, _STRIPE)])
    plsc.subcore_barrier()

    def block(i, carry):
      j0 = i * _IBC
      pltpu.sync_copy(src_hbm.at[wid, pl.ds(j0, _IBC)], sidx)
      pltpu.make_async_copy(g_hbm.at[sidx.at[0]], rows[0], sems[0]).start()
      pltpu.make_async_copy(g_hbm.at[sidx.at[1]], rows[1], sems[1]).start()
      for jj in range(_IBC):
        r = jj % 2
        pltpu.make_async_copy(g_hbm.at[sidx.at[jj]], rows[r], sems[r]).wait()
        pltpu.sync_copy(rows[r], acc.at[didx.at[j0 + jj]], add=True)
        if jj + 2 < _IBC:
          pltpu.make_async_copy(
              g_hbm.at[sidx.at[jj + 2]], rows[r], sems[r]).start()
      return carry

    lax.fori_loop(0, _CH // _IBC, block, 0)
    plsc.subcore_barrier()
    pltpu.sync_copy(acc.at[pl.ds(s * _STRIPE, _STRIPE)],
                    out_hbm.at[c, pl.ds(s * _STRIPE, _STRIPE)])

  return sc_aggregate


def _tc_encode0_body(x_ref, wn_ref, bn_ref, w1_ref, u1_ref):
  h0 = jnp.maximum(
      jnp.dot(x_ref[...], wn_ref[...], precision=_PREC) + bn_ref[...], 0.0)
  u1_ref[...] = jnp.dot(h0, w1_ref[...], precision=_PREC)


def _make_tc_encode0(interpret=False):
  return pl.pallas_call(
      _tc_encode0_body,
      grid=(_GRID,),
      in_specs=[
          pl.BlockSpec((_BR, _NODE_DIM), lambda i: (i, 0)),
          pl.BlockSpec((_NODE_DIM, _HIDDEN), lambda i: (0, 0)),
          pl.BlockSpec((1, _HIDDEN), lambda i: (0, 0)),
          pl.BlockSpec((_HIDDEN, _HIDDEN), lambda i: (0, 0)),
      ],
      out_specs=pl.BlockSpec((_BR, _HIDDEN), lambda i: (i, 0)),
      out_shape=jax.ShapeDtypeStruct((_N_NODES, _HIDDEN), jnp.float32),
      interpret=interpret,
  )


def _tc_scale_body(deg_ref, u1_ref, g1_ref, dinv_ref):
  deg = deg_ref[0, :, 0:1] + deg_ref[1, :, 0:1] + 1.0
  dinv = lax.rsqrt(jnp.maximum(deg, 1.0))
  g = dinv * u1_ref[...]
  g1_ref[...] = jnp.concatenate(
      [g, jnp.zeros((_BR, _HP - _HIDDEN), g.dtype)], axis=1)
  dinv_ref[...] = dinv


def _make_tc_scale(interpret=False):
  return pl.pallas_call(
      _tc_scale_body,
      grid=(_GRID,),
      in_specs=[
          pl.BlockSpec((_NC, _BR, _DEG_W), lambda i: (0, i, 0)),
          pl.BlockSpec((_BR, _HIDDEN), lambda i: (i, 0)),
      ],
      out_specs=[
          pl.BlockSpec((_BR, _HP), lambda i: (i, 0)),
          pl.BlockSpec((_BR, 1), lambda i: (i, 0)),
      ],
      out_shape=[
          jax.ShapeDtypeStruct((_N_NODES, _HP), jnp.float32),
          jax.ShapeDtypeStruct((_N_NODES, 1), jnp.float32),
      ],
      interpret=interpret,
  )


def _tc_layer_body(ap_ref, g_ref, dinv_ref, b_ref, w_ref, out_ref):
  a = (ap_ref[0, :, 0:_HIDDEN] + ap_ref[1, :, 0:_HIDDEN]
       + g_ref[:, 0:_HIDDEN])
  h = jnp.maximum(dinv_ref[...] * a + b_ref[...], 0.0)
  g = dinv_ref[...] * jnp.dot(h, w_ref[...], precision=_PREC)
  out_ref[...] = jnp.concatenate(
      [g, jnp.zeros((_BR, _HP - _HIDDEN), g.dtype)], axis=1)


def _make_tc_layer(interpret=False):
  return pl.pallas_call(
      _tc_layer_body,
      grid=(_GRID,),
      in_specs=[
          pl.BlockSpec((_NC, _BR, _HP), lambda i: (0, i, 0)),
          pl.BlockSpec((_BR, _HP), lambda i: (i, 0)),
          pl.BlockSpec((_BR, 1), lambda i: (i, 0)),
          pl.BlockSpec((1, _HIDDEN), lambda i: (0, 0)),
          pl.BlockSpec((_HIDDEN, _HIDDEN), lambda i: (0, 0)),
      ],
      out_specs=pl.BlockSpec((_BR, _HP), lambda i: (i, 0)),
      out_shape=jax.ShapeDtypeStruct((_N_NODES, _HP), jnp.float32),
      interpret=interpret,
  )


def _tc_head_body(ap_ref, g_ref, dinv_ref, b_ref, batch_ref,
                  wf1_ref, bf1_ref, wf2_ref, bf2_ref, out_ref, sums, counts):
  i = pl.program_id(0)

  @pl.when(i == 0)
  def _():
    sums[...] = jnp.zeros_like(sums)
    counts[...] = jnp.zeros_like(counts)

  a = (ap_ref[0, :, 0:_HIDDEN] + ap_ref[1, :, 0:_HIDDEN]
       + g_ref[:, 0:_HIDDEN])
  h3 = jnp.maximum(dinv_ref[...] * a + b_ref[...], 0.0)
  iota = lax.broadcasted_iota(jnp.int32, (1, _NUM_GRAPHS), 1).astype(jnp.float32)
  onehot = (batch_ref[...] == iota).astype(jnp.float32)
  dn = (((0,), (0,)), ((), ()))
  sums[...] += lax.dot_general(onehot, h3, dn, precision=_PREC)
  counts[...] += lax.dot_general(
      onehot, jnp.ones((_BR, 1), jnp.float32), dn, precision=_PREC)

  @pl.when(i == _GRID - 1)
  def _():
    pooled = sums[...] / jnp.maximum(counts[...], 1.0)
    z = jnp.maximum(
        jnp.dot(pooled, wf1_ref[...], precision=_PREC) + bf1_ref[...], 0.0)
    out_ref[...] = jnp.dot(z, wf2_ref[...], precision=_PREC) + bf2_ref[...]


def _make_tc_head(interpret=False):
  return pl.pallas_call(
      _tc_head_body,
      grid=(_GRID,),
      in_specs=[
          pl.BlockSpec((_NC, _BR, _HP), lambda i: (0, i, 0)),
          pl.BlockSpec((_BR, _HP), lambda i: (i, 0)),
          pl.BlockSpec((_BR, 1), lambda i: (i, 0)),
          pl.BlockSpec((1, _HIDDEN), lambda i: (0, 0)),
          pl.BlockSpec((_BR, 1), lambda i: (i, 0)),
          pl.BlockSpec((_HIDDEN, _HIDDEN), lambda i: (0, 0)),
          pl.BlockSpec((1, _HIDDEN), lambda i: (0, 0)),
          pl.BlockSpec((_HIDDEN, 1), lambda i: (0, 0)),
          pl.BlockSpec((1, 1), lambda i: (0, 0)),
      ],
      out_specs=pl.BlockSpec((_NUM_GRAPHS, 1), lambda i: (0, 0)),
      out_shape=jax.ShapeDtypeStruct((_NUM_GRAPHS, 1), jnp.float32),
      scratch_shapes=[
          pltpu.VMEM((_NUM_GRAPHS, _HIDDEN), jnp.float32),
          pltpu.VMEM((_NUM_GRAPHS, 1), jnp.float32),
      ],
      interpret=interpret,
  )


_tc_encode0 = _make_tc_encode0()
_tc_scale = _make_tc_scale()
_tc_layer = _make_tc_layer()
_tc_head = _make_tc_head()


def kernel(x, edge_index, batch, W_node, b_node, W1, b1, W2, b2, W3, b3,
           Wf1, bf1, Wf2, bf2):
  _sc_degree = _make_sc_degree()
  _sc_aggregate = _make_sc_aggregate()
  pad = _E_PAD - _N_EDGES
  pad_i = jnp.arange(pad, dtype=jnp.int32)
  src = jnp.concatenate(
      [edge_index[0].astype(jnp.int32), pad_i % _N_NODES]
  ).reshape(_NW, _CH, _K)
  dst = jnp.concatenate(
      [edge_index[1].astype(jnp.int32),
       _SINK + pad_i % (_N_ACC - _N_NODES)]
  ).reshape(_NW, _CH, _K)
  ones_deg = jnp.ones((_K, _DEG_W), jnp.float32)
  zeros_deg = jnp.zeros((_STRIPE, _DEG_W), jnp.float32)
  zeros_agg = jnp.zeros((_STRIPE, _HP), jnp.float32)
  batch_f = batch.astype(jnp.float32).reshape(_N_NODES, 1)

  deg_p = _sc_degree(dst, ones_deg, zeros_deg)
  u1 = _tc_encode0(x, W_node, b_node.reshape(1, _HIDDEN), W1)
  g1, dinv = _tc_scale(deg_p, u1)
  a1 = _sc_aggregate(src, dst, g1, zeros_agg)
  g2 = _tc_layer(a1, g1, dinv, b1.reshape(1, _HIDDEN), W2)
  a2 = _sc_aggregate(src, dst, g2, zeros_agg)
  g3 = _tc_layer(a2, g2, dinv, b2.reshape(1, _HIDDEN), W3)
  a3 = _sc_aggregate(src, dst, g3, zeros_agg)
  out = _tc_head(a3, g3, dinv, b3.reshape(1, _HIDDEN), batch_f,
                 Wf1, bf1.reshape(1, _HIDDEN), Wf2, bf2.reshape(1, 1))
  return out
